# baseline (speedup 1.0000x reference)
import functools

import jax
import jax.numpy as jnp
from jax import lax
from jax.experimental import pallas as pl
from jax.experimental.pallas import tpu as pltpu
from jax.experimental.pallas import tpu_sc as plsc

N_NODES = 10000
N_EDGES = 320000
D_FEAT = 128
HIDDEN = 64
N_OUT = 10
D2 = 16

NC, NS, L = 2, 16, 16
NW = NC * NS
EPW = N_EDGES // NW
K = 128
K_REAL = 125
NCH = EPW // K_REAL
N_PAD = 10240
RPT = N_PAD // NS
WOUT = 128
NBUF = 4

_MESH = plsc.VectorSubcoreMesh(
    core_axis_name="c", subcore_axis_name="s", num_cores=NC, num_subcores=NS
)
_SC_PARAMS = pltpu.CompilerParams(use_tc_tiling_on_sc=False)


def _zero_vmem(ref, rows, cols):
    zero = jnp.zeros((L,), jnp.float32)

    @pl.loop(0, rows)
    def _(i):
        for j in range(cols // L):
            ref[i, pl.ds(j * L, L)] = zero


@functools.partial(
    pl.kernel,
    out_type=jax.ShapeDtypeStruct((N_PAD, WOUT), jnp.float32),
    mesh=_MESH,
    compiler_params=_SC_PARAMS,
    scratch_types=[
        pltpu.VMEM((NCH, K), jnp.int32),
        pltpu.VMEM((K, D2), jnp.float32),
        pltpu.VMEM((RPT, D2), jnp.float32),
        pltpu.VMEM_SHARED((N_PAD, D2), jnp.float32),
    ],
)
def _deg_kernel(dst_hbm, out_hbm, dst_v, ones_v, stage_v, acc):
    cid = lax.axis_index("c")
    sid = lax.axis_index("s")
    wid = cid * NS + sid
    _zero_vmem(stage_v, RPT, D2)
    pltpu.sync_copy(stage_v, acc.at[pl.ds(sid * RPT, RPT)])

    one = jnp.ones((L,), jnp.float32)

    @pl.loop(0, K)
    def _(i):
        ones_v[i, :] = one

    pltpu.sync_copy(dst_hbm.at[wid], dst_v)
    plsc.subcore_barrier()

    @pl.loop(0, NCH)
    def _(j):
        pltpu.sync_copy(ones_v, acc.at[dst_v.at[j]], add=True)

    plsc.subcore_barrier()
    pltpu.sync_copy(acc.at[pl.ds(sid * RPT, RPT)], stage_v)
    pltpu.sync_copy(
        stage_v, out_hbm.at[pl.ds(sid * RPT, RPT), pl.ds(cid * HIDDEN, D2)]
    )


def _make_agg(D, nbuf):
    @functools.partial(
        pl.kernel,
        out_type=jax.ShapeDtypeStruct((N_PAD, WOUT), jnp.float32),
        mesh=_MESH,
        compiler_params=_SC_PARAMS,
        scratch_types=[
            pltpu.VMEM((NCH, K), jnp.int32),
            pltpu.VMEM((NCH, K), jnp.int32),
            pltpu.VMEM((nbuf, K, D), jnp.float32),
            pltpu.VMEM((RPT, D), jnp.float32),
            pltpu.VMEM_SHARED((N_PAD, D), jnp.float32),
            pltpu.SemaphoreType.DMA((nbuf,)),
            pltpu.SemaphoreType.DMA((nbuf,)),
        ],
    )
    def agg(
        g_hbm, src_hbm, dst_hbm, out_hbm, src_v, dst_v, rows_v, stage_v, acc,
        gsems, ssems,
    ):
        cid = lax.axis_index("c")
        sid = lax.axis_index("s")
        wid = cid * NS + sid
        _zero_vmem(stage_v, RPT, D)
        pltpu.sync_copy(stage_v, acc.at[pl.ds(sid * RPT, RPT)])
        pltpu.sync_copy(src_hbm.at[wid], src_v)
        pltpu.sync_copy(dst_hbm.at[wid], dst_v)
        plsc.subcore_barrier()

        if nbuf == 2:
            for b in range(2):
                pltpu.async_copy(g_hbm.at[src_v.at[b]], rows_v.at[b], gsems.at[b])

            @pl.loop(0, NCH, step=2)
            def _(j):
                for b in range(2):
                    jj = j + b
                    pltpu.make_async_copy(
                        g_hbm.at[src_v.at[jj]], rows_v.at[b], gsems.at[b]
                    ).wait()
                    pltpu.sync_copy(rows_v.at[b], acc.at[dst_v.at[jj]], add=True)

                    @pl.when(jj + 2 < NCH)
                    def _():
                        pltpu.async_copy(
                            g_hbm.at[src_v.at[jj + 2]], rows_v.at[b], gsems.at[b]
                        )
        else:
            for b in range(2):
                pltpu.async_copy(g_hbm.at[src_v.at[b]], rows_v.at[b], gsems.at[b])

            @pl.loop(0, NCH, step=4)
            def _(j):
                for b in range(4):
                    jj = j + b
                    bn = (b + 2) % 4
                    pltpu.make_async_copy(
                        g_hbm.at[src_v.at[jj]], rows_v.at[b], gsems.at[b]
                    ).wait()

                    @pl.when(jj >= 2)
                    def _():
                        pltpu.make_async_copy(
                            rows_v.at[bn], acc.at[dst_v.at[jj]], ssems.at[bn]
                        ).wait()

                    @pl.when(jj + 2 < NCH)
                    def _():
                        pltpu.async_copy(
                            g_hbm.at[src_v.at[jj + 2]], rows_v.at[bn], gsems.at[bn]
                        )

                    pltpu.async_copy(
                        rows_v.at[b], acc.at[dst_v.at[jj]], ssems.at[b], add=True
                    )

            for jj in (NCH - 2, NCH - 1):
                b = jj % 4
                pltpu.make_async_copy(
                    rows_v.at[b], acc.at[dst_v.at[jj]], ssems.at[b]
                ).wait()

        plsc.subcore_barrier()
        pltpu.sync_copy(acc.at[pl.ds(sid * RPT, RPT)], stage_v)
        pltpu.sync_copy(
            stage_v, out_hbm.at[pl.ds(sid * RPT, RPT), pl.ds(cid * HIDDEN, D)]
        )

    return agg


_agg64 = _make_agg(HIDDEN, 2)
_agg16 = _make_agg(D2, 4)


ROWS = 5000
GRID = N_NODES // ROWS


def _dinv(d_ref):
    deg = d_ref[:, 0:1] + d_ref[:, HIDDEN : HIDDEN + 1] + 1.0
    return lax.rsqrt(deg)


def _tc1_body(x_ref, w_ref, d_ref, o_ref):
    o_ref[...] = (x_ref[...] @ w_ref[...]) * _dinv(d_ref)


def _tc2_body(s_ref, g1_ref, d_ref, w_ref, b_ref, o_ref):
    dinv = _dinv(d_ref)
    s = s_ref[:, :HIDDEN] + s_ref[:, HIDDEN:]
    z = (s + g1_ref[...]) * dinv + b_ref[...]
    z = jnp.maximum(z, 0.0)
    o_ref[...] = (z @ w_ref[...]) * dinv


def _tc3_body(t_ref, g2_ref, d_ref, b_ref, o_ref):
    dinv = _dinv(d_ref)
    t = t_ref[:, :D2] + t_ref[:, HIDDEN : HIDDEN + D2]
    v = (t + g2_ref[...]) * dinv + b_ref[...]
    col = lax.broadcasted_iota(jnp.int32, v.shape, 1)
    vm = jnp.where(col < N_OUT, v, -jnp.inf)
    m = jnp.max(vm, axis=1, keepdims=True)
    lse = jnp.log(jnp.sum(jnp.exp(vm - m), axis=1, keepdims=True))
    o_ref[...] = (v - m - lse)[:, :N_OUT]


def _row_spec(d):
    return pl.BlockSpec((ROWS, d), lambda i: (i, 0))


def _full_spec(r, c):
    return pl.BlockSpec((r, c), lambda i: (0, 0))


_tc1 = pl.pallas_call(
    _tc1_body,
    grid=(GRID,),
    in_specs=[
        _row_spec(D_FEAT),
        _full_spec(D_FEAT, HIDDEN),
        _row_spec(WOUT),
    ],
    out_specs=_row_spec(HIDDEN),
    out_shape=jax.ShapeDtypeStruct((N_NODES, HIDDEN), jnp.float32),
)

_tc2 = pl.pallas_call(
    _tc2_body,
    grid=(GRID,),
    in_specs=[
        _row_spec(WOUT),
        _row_spec(HIDDEN),
        _row_spec(WOUT),
        _full_spec(HIDDEN, D2),
        _full_spec(1, HIDDEN),
    ],
    out_specs=_row_spec(D2),
    out_shape=jax.ShapeDtypeStruct((N_NODES, D2), jnp.float32),
)

_tc3 = pl.pallas_call(
    _tc3_body,
    grid=(GRID,),
    in_specs=[
        _row_spec(WOUT),
        _row_spec(D2),
        _row_spec(WOUT),
        _full_spec(1, D2),
    ],
    out_specs=_row_spec(N_OUT),
    out_shape=jax.ShapeDtypeStruct((N_NODES, N_OUT), jnp.float32),
)


def kernel(x, edge_index, W1, b1, W2, b2):
    ei = edge_index.astype(jnp.int32)
    pad = ((0, 0), (0, 0), (0, K - K_REAL))
    src = jnp.pad(ei[0].reshape(NW, NCH, K_REAL), pad)
    dst = jnp.pad(ei[1].reshape(NW, NCH, K_REAL), pad, constant_values=N_PAD - 1)
    w2p = jnp.pad(W2, ((0, 0), (0, D2 - N_OUT)))
    b1r = b1.reshape(1, HIDDEN)
    b2p = jnp.pad(b2, (0, D2 - N_OUT)).reshape(1, D2)

    degp = _deg_kernel(dst)
    g1 = _tc1(x, W1, degp)
    s = _agg64(g1, src, dst)
    g2 = _tc2(s, g1, degp, w2p, b1r)
    t = _agg16(g2, src, dst)
    return _tc3(t, g2, degp, b2p)


# speedup vs baseline: 1.6372x; 1.6372x over previous
import functools

import jax
import jax.numpy as jnp
from jax import lax
from jax.experimental import pallas as pl
from jax.experimental.pallas import tpu as pltpu
from jax.experimental.pallas import tpu_sc as plsc

N_NODES = 10000
N_EDGES = 320000
D_FEAT = 128
HIDDEN = 64
N_OUT = 10
D2 = 16

NC, NS, L = 2, 16, 16
NW = NC * NS
EPW = N_EDGES // NW
K = 125
NCH = EPW // K
N_PAD = 10240
RPT = N_PAD // NS
WOUT = 128
NBUF = 4

_MESH = plsc.VectorSubcoreMesh(
    core_axis_name="c", subcore_axis_name="s", num_cores=NC, num_subcores=NS
)
_SC_PARAMS = pltpu.CompilerParams(use_tc_tiling_on_sc=False)


def _zero_vmem(ref, rows, cols):
    zero = jnp.zeros((L,), jnp.float32)

    @pl.loop(0, rows)
    def _(i):
        for j in range(cols // L):
            ref[i, pl.ds(j * L, L)] = zero


@functools.partial(
    pl.kernel,
    out_type=jax.ShapeDtypeStruct((N_PAD, WOUT), jnp.float32),
    mesh=_MESH,
    compiler_params=_SC_PARAMS,
    scratch_types=[
        pltpu.VMEM((NCH, K), jnp.int32),
        pltpu.VMEM((K, D2), jnp.float32),
        pltpu.VMEM((RPT, D2), jnp.float32),
        pltpu.VMEM_SHARED((N_PAD, D2), jnp.float32),
    ],
)
def _deg_kernel(dst_hbm, out_hbm, dst_v, ones_v, stage_v, acc):
    cid = lax.axis_index("c")
    sid = lax.axis_index("s")
    wid = cid * NS + sid
    _zero_vmem(stage_v, RPT, D2)
    pltpu.sync_copy(stage_v, acc.at[pl.ds(sid * RPT, RPT)])

    one = jnp.ones((L,), jnp.float32)

    @pl.loop(0, K)
    def _(i):
        ones_v[i, :] = one

    pltpu.sync_copy(dst_hbm.at[wid], dst_v)
    plsc.subcore_barrier()

    @pl.loop(0, NCH)
    def _(j):
        pltpu.sync_copy(ones_v, acc.at[dst_v.at[j]], add=True)

    plsc.subcore_barrier()
    pltpu.sync_copy(acc.at[pl.ds(sid * RPT, RPT)], stage_v)
    pltpu.sync_copy(
        stage_v, out_hbm.at[pl.ds(sid * RPT, RPT), pl.ds(cid * HIDDEN, D2)]
    )


def _make_agg(D, nbuf):
    @functools.partial(
        pl.kernel,
        out_type=jax.ShapeDtypeStruct((N_PAD, WOUT), jnp.float32),
        mesh=_MESH,
        compiler_params=_SC_PARAMS,
        scratch_types=[
            pltpu.VMEM((NCH, K), jnp.int32),
            pltpu.VMEM((NCH, K), jnp.int32),
            pltpu.VMEM((nbuf, K, D), jnp.float32),
            pltpu.VMEM((RPT, D), jnp.float32),
            pltpu.VMEM_SHARED((N_PAD, D), jnp.float32),
            pltpu.SemaphoreType.DMA((nbuf,)),
            pltpu.SemaphoreType.DMA((nbuf,)),
        ],
    )
    def agg(
        g_hbm, src_hbm, dst_hbm, out_hbm, src_v, dst_v, rows_v, stage_v, acc,
        gsems, ssems,
    ):
        cid = lax.axis_index("c")
        sid = lax.axis_index("s")
        wid = cid * NS + sid
        _zero_vmem(stage_v, RPT, D)
        pltpu.sync_copy(stage_v, acc.at[pl.ds(sid * RPT, RPT)])
        pltpu.sync_copy(src_hbm.at[wid], src_v)
        pltpu.sync_copy(dst_hbm.at[wid], dst_v)
        plsc.subcore_barrier()

        if nbuf == 2:
            for b in range(2):
                pltpu.async_copy(g_hbm.at[src_v.at[b]], rows_v.at[b], gsems.at[b])

            @pl.loop(0, NCH, step=2)
            def _(j):
                for b in range(2):
                    jj = j + b
                    pltpu.make_async_copy(
                        g_hbm.at[src_v.at[jj]], rows_v.at[b], gsems.at[b]
                    ).wait()
                    pltpu.sync_copy(rows_v.at[b], acc.at[dst_v.at[jj]], add=True)

                    @pl.when(jj + 2 < NCH)
                    def _():
                        pltpu.async_copy(
                            g_hbm.at[src_v.at[jj + 2]], rows_v.at[b], gsems.at[b]
                        )
        else:
            for b in range(2):
                pltpu.async_copy(g_hbm.at[src_v.at[b]], rows_v.at[b], gsems.at[b])

            @pl.loop(0, NCH, step=4)
            def _(j):
                for b in range(4):
                    jj = j + b
                    bn = (b + 2) % 4
                    pltpu.make_async_copy(
                        g_hbm.at[src_v.at[jj]], rows_v.at[b], gsems.at[b]
                    ).wait()

                    @pl.when(jj >= 2)
                    def _():
                        pltpu.make_async_copy(
                            rows_v.at[bn], acc.at[dst_v.at[jj]], ssems.at[bn]
                        ).wait()

                    @pl.when(jj + 2 < NCH)
                    def _():
                        pltpu.async_copy(
                            g_hbm.at[src_v.at[jj + 2]], rows_v.at[bn], gsems.at[bn]
                        )

                    pltpu.async_copy(
                        rows_v.at[b], acc.at[dst_v.at[jj]], ssems.at[b], add=True
                    )

            for jj in (NCH - 2, NCH - 1):
                b = jj % 4
                pltpu.make_async_copy(
                    rows_v.at[b], acc.at[dst_v.at[jj]], ssems.at[b]
                ).wait()

        plsc.subcore_barrier()
        pltpu.sync_copy(acc.at[pl.ds(sid * RPT, RPT)], stage_v)
        pltpu.sync_copy(
            stage_v, out_hbm.at[pl.ds(sid * RPT, RPT), pl.ds(cid * HIDDEN, D)]
        )

    return agg


_agg64 = _make_agg(HIDDEN, 2)
_agg16 = _make_agg(D2, 4)


ROWS = 5000
GRID = N_NODES // ROWS


def _dinv(d_ref):
    deg = d_ref[:, 0:1] + d_ref[:, HIDDEN : HIDDEN + 1] + 1.0
    return lax.rsqrt(deg)


def _tc1_body(x_ref, w_ref, d_ref, o_ref):
    o_ref[...] = (x_ref[...] @ w_ref[...]) * _dinv(d_ref)


def _tc2_body(s_ref, g1_ref, d_ref, w_ref, b_ref, o_ref):
    dinv = _dinv(d_ref)
    s = s_ref[:, :HIDDEN] + s_ref[:, HIDDEN:]
    z = (s + g1_ref[...]) * dinv + b_ref[...]
    z = jnp.maximum(z, 0.0)
    o_ref[...] = (z @ w_ref[...]) * dinv


def _tc3_body(t_ref, g2_ref, d_ref, b_ref, o_ref):
    dinv = _dinv(d_ref)
    t = t_ref[:, :D2] + t_ref[:, HIDDEN : HIDDEN + D2]
    v = (t + g2_ref[...]) * dinv + b_ref[...]
    col = lax.broadcasted_iota(jnp.int32, v.shape, 1)
    vm = jnp.where(col < N_OUT, v, -jnp.inf)
    m = jnp.max(vm, axis=1, keepdims=True)
    lse = jnp.log(jnp.sum(jnp.exp(vm - m), axis=1, keepdims=True))
    o_ref[...] = (v - m - lse)[:, :N_OUT]


def _row_spec(d):
    return pl.BlockSpec((ROWS, d), lambda i: (i, 0))


def _full_spec(r, c):
    return pl.BlockSpec((r, c), lambda i: (0, 0))


_tc1 = pl.pallas_call(
    _tc1_body,
    grid=(GRID,),
    in_specs=[
        _row_spec(D_FEAT),
        _full_spec(D_FEAT, HIDDEN),
        _row_spec(WOUT),
    ],
    out_specs=_row_spec(HIDDEN),
    out_shape=jax.ShapeDtypeStruct((N_NODES, HIDDEN), jnp.float32),
)

_tc2 = pl.pallas_call(
    _tc2_body,
    grid=(GRID,),
    in_specs=[
        _row_spec(WOUT),
        _row_spec(HIDDEN),
        _row_spec(WOUT),
        _full_spec(HIDDEN, D2),
        _full_spec(1, HIDDEN),
    ],
    out_specs=_row_spec(D2),
    out_shape=jax.ShapeDtypeStruct((N_NODES, D2), jnp.float32),
)

_tc3 = pl.pallas_call(
    _tc3_body,
    grid=(GRID,),
    in_specs=[
        _row_spec(WOUT),
        _row_spec(D2),
        _row_spec(WOUT),
        _full_spec(1, D2),
    ],
    out_specs=_row_spec(N_OUT),
    out_shape=jax.ShapeDtypeStruct((N_NODES, N_OUT), jnp.float32),
)


def kernel(x, edge_index, W1, b1, W2, b2):
    ei = edge_index.astype(jnp.int32)
    src = ei[0].reshape(NW, NCH, K)
    dst = ei[1].reshape(NW, NCH, K)
    w2p = jnp.pad(W2, ((0, 0), (0, D2 - N_OUT)))
    b1r = b1.reshape(1, HIDDEN)
    b2p = jnp.pad(b2, (0, D2 - N_OUT)).reshape(1, D2)

    degp = _deg_kernel(dst)
    g1 = _tc1(x, W1, degp)
    s = _agg64(g1, src, dst)
    g2 = _tc2(s, g1, degp, w2p, b1r)
    t = _agg16(g2, src, dst)
    return _tc3(t, g2, degp, b2p)
